# Initial kernel scaffold; baseline (speedup 1.0000x reference)
import jax
import jax.numpy as jnp
from jax.experimental import pallas as pl


def kernel(inputs, W1, b1, W2, b2, W3, b3, Wd1, bd1, Wd2, bd2):
    raise NotImplementedError("write your pallas kernel here")


# speedup vs baseline: 2.9040x; 2.9040x over previous
import functools

import jax
import jax.numpy as jnp
from jax.experimental import pallas as pl
from jax.experimental.pallas import tpu as pltpu

B, N, F = 1024, 100, 16
K = 3
H = 32
D = 64
J = 8

_DOT = functools.partial(
    jnp.dot, precision=jax.lax.Precision.HIGHEST,
    preferred_element_type=jnp.float32)


def _relu(x):
    return jnp.maximum(x, 0.0)


def _tc_body(eta_ref, phi_ref, x_ref, W1c_ref, W1b_ref, b1_ref, W2_ref,
             b2_ref, W3_ref, b3_ref, Wd1_ref, bd1_ref, Wd2_ref, bd2_ref,
             out_ref):
    eta = eta_ref[...]
    phi = phi_ref[...]
    de = eta[:, :, None] - eta[:, None, :]
    dp = phi[:, :, None] - phi[:, None, :]
    d2 = de * de + dp * dp
    iota_r = jax.lax.broadcasted_iota(jnp.int32, (J, N, N), 1)
    iota_c = jax.lax.broadcasted_iota(jnp.int32, (J, N, N), 2)
    d2 = d2 + jnp.where(iota_r == iota_c, jnp.float32(1e9), jnp.float32(0.0))

    onehots = []
    for _ in range(K):
        mval = jnp.min(d2, axis=-1, keepdims=True)
        cand = jnp.where(d2 == mval, iota_c, jnp.int32(N))
        imin = jnp.min(cand, axis=-1, keepdims=True)
        fo = iota_c == imin
        onehots.append(fo)
        d2 = jnp.where(fo, jnp.float32(jnp.inf), d2)

    x = x_ref[...]
    W1c = W1c_ref[...]
    W1b = W1b_ref[...]
    b1 = b1_ref[...]
    W2 = W2_ref[...]
    b2 = b2_ref[...]
    W3 = W3_ref[...]
    b3 = b3_ref[...]
    Wd1 = Wd1_ref[...]
    bd1 = bd1_ref[...]
    Wd2 = Wd2_ref[...]
    bd2 = bd2_ref[...]

    for j in range(J):
        xj = x[j]
        A = _DOT(xj, W1c)
        Bv = _DOT(xj, W1b)
        pt_sum = jnp.zeros((N, H), jnp.float32)
        for k in range(K):
            oh = onehots[k][j].astype(jnp.float32)
            g = _DOT(oh, Bv)
            h = _relu(A + g + b1)
            h = _relu(_DOT(h, W2) + b2)
            h = _relu(_DOT(h, W3) + b3)
            pt_sum = pt_sum + h
        pt = pt_sum * jnp.float32(1.0 / K)
        pooled = jnp.sum(pt, axis=0, keepdims=True) * jnp.float32(1.0 / N)
        o = _relu(_DOT(pooled, Wd1) + bd1)
        o = _relu(_DOT(o, Wd2) + bd2)
        out_ref[j, :] = o[0]


def kernel(inputs, W1, b1, W2, b2, W3, b3, Wd1, bd1, Wd2, bd2):
    eta = inputs[:, :, 1]
    phi = inputs[:, :, 2]
    W1c = W1[:F] - W1[F:]
    W1b = W1[F:]
    full = lambda shape: pl.BlockSpec(shape, lambda i: (0,) * len(shape))
    out = pl.pallas_call(
        _tc_body,
        grid=(B // J,),
        in_specs=[
            pl.BlockSpec((J, N), lambda i: (i, 0)),
            pl.BlockSpec((J, N), lambda i: (i, 0)),
            pl.BlockSpec((J, N, F), lambda i: (i, 0, 0)),
            full((F, H)), full((F, H)), full((1, H)),
            full((H, H)), full((1, H)),
            full((H, H)), full((1, H)),
            full((H, D)), full((1, D)),
            full((D, D)), full((1, D)),
        ],
        out_specs=pl.BlockSpec((J, D), lambda i: (i, 0)),
        out_shape=jax.ShapeDtypeStruct((B, D), jnp.float32),
        compiler_params=pltpu.CompilerParams(
            dimension_semantics=("arbitrary",)),
    )(eta, phi, inputs, W1c, W1b, b1.reshape(1, H), W2, b2.reshape(1, H),
      W3, b3.reshape(1, H), Wd1, bd1.reshape(1, D), Wd2, bd2.reshape(1, D))
    return out


# speedup vs baseline: 7.8371x; 2.6987x over previous
import functools

import jax
import jax.numpy as jnp
from jax.experimental import pallas as pl
from jax.experimental.pallas import tpu as pltpu

B, N, F = 1024, 100, 16
K = 3
H = 32
D = 64
J = 8

_DOT = functools.partial(
    jnp.dot, precision=jax.lax.Precision.DEFAULT,
    preferred_element_type=jnp.float32)


def _relu(x):
    return jnp.maximum(x, 0.0)


def _tc_body(eta_ref, phi_ref, x_ref, W1c_ref, W1b_ref, b1_ref, W2_ref,
             b2_ref, W3_ref, b3_ref, Wd1_ref, bd1_ref, Wd2_ref, bd2_ref,
             out_ref):
    eta = eta_ref[...]
    phi = phi_ref[...]
    de = eta[:, :, None] - eta[:, None, :]
    dp = phi[:, :, None] - phi[:, None, :]
    d2 = de * de + dp * dp
    iota_r = jax.lax.broadcasted_iota(jnp.int32, (J, N, N), 1)
    iota_c = jax.lax.broadcasted_iota(jnp.int32, (J, N, N), 2)
    d2 = d2 + jnp.where(iota_r == iota_c, jnp.float32(1e9), jnp.float32(0.0))

    onehots = []
    for _ in range(K):
        mval = jnp.min(d2, axis=-1, keepdims=True)
        cand = jnp.where(d2 == mval, iota_c, jnp.int32(N))
        imin = jnp.min(cand, axis=-1, keepdims=True)
        fo = iota_c == imin
        onehots.append(fo)
        d2 = jnp.where(fo, jnp.float32(jnp.inf), d2)

    x = x_ref[...]
    W1c = W1c_ref[...]
    W1b = W1b_ref[...]
    b1 = b1_ref[...]
    W2 = W2_ref[...]
    b2 = b2_ref[...]
    W3 = W3_ref[...]
    b3 = b3_ref[...]
    Wd1 = Wd1_ref[...]
    bd1 = bd1_ref[...]
    Wd2 = Wd2_ref[...]
    bd2 = bd2_ref[...]

    for j in range(J):
        xj = x[j]
        A = _DOT(xj, W1c)
        Bv = _DOT(xj, W1b)
        pt_sum = jnp.zeros((N, H), jnp.float32)
        for k in range(K):
            oh = onehots[k][j].astype(jnp.float32)
            g = _DOT(oh, Bv)
            h = _relu(A + g + b1)
            h = _relu(_DOT(h, W2) + b2)
            h = _relu(_DOT(h, W3) + b3)
            pt_sum = pt_sum + h
        pt = pt_sum * jnp.float32(1.0 / K)
        pooled = jnp.sum(pt, axis=0, keepdims=True) * jnp.float32(1.0 / N)
        o = _relu(_DOT(pooled, Wd1) + bd1)
        o = _relu(_DOT(o, Wd2) + bd2)
        out_ref[j, :] = o[0]


def kernel(inputs, W1, b1, W2, b2, W3, b3, Wd1, bd1, Wd2, bd2):
    eta = inputs[:, :, 1]
    phi = inputs[:, :, 2]
    W1c = W1[:F] - W1[F:]
    W1b = W1[F:]
    full = lambda shape: pl.BlockSpec(shape, lambda i: (0,) * len(shape))
    out = pl.pallas_call(
        _tc_body,
        grid=(B // J,),
        in_specs=[
            pl.BlockSpec((J, N), lambda i: (i, 0)),
            pl.BlockSpec((J, N), lambda i: (i, 0)),
            pl.BlockSpec((J, N, F), lambda i: (i, 0, 0)),
            full((F, H)), full((F, H)), full((1, H)),
            full((H, H)), full((1, H)),
            full((H, H)), full((1, H)),
            full((H, D)), full((1, D)),
            full((D, D)), full((1, D)),
        ],
        out_specs=pl.BlockSpec((J, D), lambda i: (i, 0)),
        out_shape=jax.ShapeDtypeStruct((B, D), jnp.float32),
        compiler_params=pltpu.CompilerParams(
            dimension_semantics=("arbitrary",)),
    )(eta, phi, inputs, W1c, W1b, b1.reshape(1, H), W2, b2.reshape(1, H),
      W3, b3.reshape(1, H), Wd1, bd1.reshape(1, D), Wd2, bd2.reshape(1, D))
    return out


# speedup vs baseline: 8.2944x; 1.0583x over previous
import functools

import jax
import jax.numpy as jnp
from jax.experimental import pallas as pl
from jax.experimental.pallas import tpu as pltpu

B, N, F = 1024, 100, 16
K = 3
H = 32
D = 64
J = 16

_DOT = functools.partial(
    jnp.dot, precision=jax.lax.Precision.DEFAULT,
    preferred_element_type=jnp.float32)


def _relu(x):
    return jnp.maximum(x, 0.0)


def _tc_body(eta_ref, phi_ref, x_ref, W1c_ref, W1b_ref, b1_ref, W2_ref,
             b2_ref, W3_ref, b3_ref, Wd1_ref, bd1_ref, Wd2_ref, bd2_ref,
             out_ref):
    eta = eta_ref[...]
    phi = phi_ref[...]
    de = eta[:, :, None] - eta[:, None, :]
    dp = phi[:, :, None] - phi[:, None, :]
    d2 = de * de + dp * dp
    iota_r = jax.lax.broadcasted_iota(jnp.int32, (J, N, N), 1)
    iota_c = jax.lax.broadcasted_iota(jnp.int32, (J, N, N), 2)
    d2 = d2 + jnp.where(iota_r == iota_c, jnp.float32(1e9), jnp.float32(0.0))
    iota_f = iota_c.astype(jnp.float32)

    onehots = []
    for _ in range(K):
        mval = jnp.min(d2, axis=-1, keepdims=True)
        cand = jnp.where(d2 == mval, iota_f, jnp.float32(N))
        imin = jnp.min(cand, axis=-1, keepdims=True)
        fo = iota_f == imin
        onehots.append(fo)
        d2 = jnp.where(fo, jnp.float32(jnp.inf), d2)

    x = x_ref[...]
    W1c = W1c_ref[...]
    W1b = W1b_ref[...]
    b1 = b1_ref[...]
    W2 = W2_ref[...]
    b2 = b2_ref[...]
    W3 = W3_ref[...]
    b3 = b3_ref[...]
    Wd1 = Wd1_ref[...]
    bd1 = bd1_ref[...]
    Wd2 = Wd2_ref[...]
    bd2 = bd2_ref[...]

    for j in range(J):
        xj = x[j]
        A = _DOT(xj, W1c)
        Bv = _DOT(xj, W1b)
        pt_sum = jnp.zeros((N, H), jnp.float32)
        for k in range(K):
            oh = onehots[k][j].astype(jnp.float32)
            g = _DOT(oh, Bv)
            h = _relu(A + g + b1)
            h = _relu(_DOT(h, W2) + b2)
            h = _relu(_DOT(h, W3) + b3)
            pt_sum = pt_sum + h
        pt = pt_sum * jnp.float32(1.0 / K)
        pooled = jnp.sum(pt, axis=0, keepdims=True) * jnp.float32(1.0 / N)
        o = _relu(_DOT(pooled, Wd1) + bd1)
        o = _relu(_DOT(o, Wd2) + bd2)
        out_ref[j, :] = o[0]


def kernel(inputs, W1, b1, W2, b2, W3, b3, Wd1, bd1, Wd2, bd2):
    eta = inputs[:, :, 1]
    phi = inputs[:, :, 2]
    W1c = W1[:F] - W1[F:]
    W1b = W1[F:]
    full = lambda shape: pl.BlockSpec(shape, lambda i: (0,) * len(shape))
    out = pl.pallas_call(
        _tc_body,
        grid=(B // J,),
        in_specs=[
            pl.BlockSpec((J, N), lambda i: (i, 0)),
            pl.BlockSpec((J, N), lambda i: (i, 0)),
            pl.BlockSpec((J, N, F), lambda i: (i, 0, 0)),
            full((F, H)), full((F, H)), full((1, H)),
            full((H, H)), full((1, H)),
            full((H, H)), full((1, H)),
            full((H, D)), full((1, D)),
            full((D, D)), full((1, D)),
        ],
        out_specs=pl.BlockSpec((J, D), lambda i: (i, 0)),
        out_shape=jax.ShapeDtypeStruct((B, D), jnp.float32),
        compiler_params=pltpu.CompilerParams(
            dimension_semantics=("arbitrary",)),
    )(eta, phi, inputs, W1c, W1b, b1.reshape(1, H), W2, b2.reshape(1, H),
      W3, b3.reshape(1, H), Wd1, bd1.reshape(1, D), Wd2, bd2.reshape(1, D))
    return out
